# Initial kernel scaffold; baseline (speedup 1.0000x reference)
import jax
import jax.numpy as jnp
from jax.experimental import pallas as pl


def kernel(edges, edge_indices, node_to_graph_idx, num_graphs, W0, b0, W1, b1, W2, b2):
    raise NotImplementedError("write your pallas kernel here")


# speedup vs baseline: 4.3299x; 4.3299x over previous
import jax
import jax.numpy as jnp
from jax import lax
from jax.experimental import pallas as pl
from jax.experimental.pallas import tpu as pltpu
from jax.experimental.pallas import tpu_sc as plsc

NC = 2
NS = 16
L = 16

E = 320000
N = 10000
D = 128
G = 1024

HG = G // NC
EPT = E // NS
SCH = 800
RCH = 64

_BCAST_DNUMS = lax.GatherDimensionNumbers(
    offset_dims=(), collapsed_slice_dims=(0,), start_index_map=(0,))


def _bcast_lane(vec, j):
    idx = jnp.full((L, 1), j, jnp.int32)
    return lax.gather(vec, idx, _BCAST_DNUMS, slice_sizes=(1,),
                      mode=lax.GatherScatterMode.PROMISE_IN_BOUNDS)


def _sc_body(edges_hbm, src_hbm, n2g_hbm, part_hbm,
             node_v, idx_c, ids_c, sbuf, rows_v, acc, sem):
    c = lax.axis_index("c")
    s = lax.axis_index("s")
    iota = lax.iota(jnp.int32, L)
    col_idx = [iota + kk * L for kk in range(D // L)]

    pltpu.sync_copy(n2g_hbm, node_v)

    neg_inf = jnp.full((L,), -jnp.inf, jnp.float32)

    def init_row(r, _):
        for kk in range(D // L):
            acc[r, pl.ds(kk * L, L)] = neg_inf
        return 0

    lax.fori_loop(0, HG, init_row, 0)

    ebase = s * EPT
    cgbase = c * HG

    def chunk_a(i, k):
        base = ebase + i * SCH
        pltpu.sync_copy(src_hbm.at[pl.ds(base, SCH)], sbuf)

        def group(g, k):
            src_vec = sbuf[pl.ds(g * L, L)]
            ids = plsc.load_gather(node_v, [src_vec])
            ids = jnp.minimum(ids, G - 1) - cgbase
            m = (ids >= 0) & (ids < HG)
            mi = m.astype(jnp.int32)
            evec = jnp.full((L,), base + g * L, jnp.int32) + iota
            pos = k - 1 + plsc.cumsum(mi)
            plsc.store_scatter(idx_c, [pos], evec, mask=m)
            plsc.store_scatter(ids_c, [pos], ids, mask=m)
            return k + jnp.sum(mi)

        return lax.fori_loop(0, SCH // L, group, k)

    k = lax.fori_loop(0, EPT // SCH, chunk_a, jnp.int32(0))

    pad_ids = jnp.full((L,), HG, jnp.int32)
    zeros = jnp.zeros((L,), jnp.int32)
    for t in range(RCH // L):
        pos = k + t * L + iota
        plsc.store_scatter(idx_c, [pos], zeros)
        plsc.store_scatter(ids_c, [pos], pad_ids)
    kpad = jnp.bitwise_and(k + (RCH - 1), -RCH)

    def chunk_b(i, _):
        cb = i * RCH
        pltpu.async_copy(edges_hbm.at[idx_c.at[pl.ds(cb, RCH)]],
                         rows_v, sem).wait()

        def group(mq, _):
            idvec = ids_c[pl.ds(cb + mq * L, L)]
            for j in range(L):
                gj = _bcast_lane(idvec, j)
                row = mq * L + j
                for kk in range(D // L):
                    a = plsc.load_gather(acc, [gj, col_idx[kk]])
                    e = rows_v[row, pl.ds(kk * L, L)]
                    plsc.store_scatter(acc, [gj, col_idx[kk]],
                                       jnp.maximum(a, e))
            return 0

        lax.fori_loop(0, RCH // L, group, 0)
        return 0

    lax.fori_loop(0, jnp.right_shift(kpad, 6), chunk_b, 0)

    pltpu.sync_copy(acc.at[pl.ds(0, HG), :],
                    part_hbm.at[s, pl.ds(cgbase, HG), :])


def _segment_max_partials(edges, src, n2g):
    mesh = plsc.VectorSubcoreMesh(core_axis_name="c", subcore_axis_name="s")
    fn = pl.kernel(
        _sc_body,
        out_type=jax.ShapeDtypeStruct((NS, G, D), jnp.float32),
        mesh=mesh,
        scratch_types=[
            pltpu.VMEM((N,), jnp.int32),
            pltpu.VMEM((EPT + RCH + L,), jnp.int32),
            pltpu.VMEM((EPT + RCH + L,), jnp.int32),
            pltpu.VMEM((SCH,), jnp.int32),
            pltpu.VMEM((RCH, D), jnp.float32),
            pltpu.VMEM((HG + 8, D), jnp.float32),
            pltpu.SemaphoreType.DMA,
        ],
        compiler_params=pltpu.CompilerParams(needs_layout_passes=False),
    )
    return fn(edges, src, n2g)


def _mlp_body(p_ref, w0_ref, b0_ref, w1_ref, b1_ref, w2_ref, b2_ref, out_ref):
    x = jnp.max(p_ref[...], axis=0)
    x = jnp.where(x == -jnp.inf, 0.0, x)
    h = jnp.dot(x, w0_ref[...], preferred_element_type=jnp.float32,
                precision="highest") + b0_ref[...]
    h = jnp.maximum(h, 0.0)
    h = jnp.dot(h, w1_ref[...], preferred_element_type=jnp.float32,
                precision="highest") + b1_ref[...]
    h = jnp.maximum(h, 0.0)
    out_ref[...] = jnp.dot(h, w2_ref[...], preferred_element_type=jnp.float32,
                           precision="highest") + b2_ref[...]


def _mlp(partials, W0, b0, W1, b1, W2, b2):
    return pl.pallas_call(
        _mlp_body,
        out_shape=jax.ShapeDtypeStruct((G, W2.shape[1]), jnp.float32),
    )(partials, W0, b0.reshape(1, -1), W1, b1.reshape(1, -1),
      W2, b2.reshape(1, -1))


def kernel(edges, edge_indices, node_to_graph_idx, num_graphs,
           W0, b0, W1, b1, W2, b2):
    del num_graphs
    src = edge_indices[0]
    partials = _segment_max_partials(edges, src, node_to_graph_idx)
    return _mlp(partials, W0, b0, W1, b1, W2, b2)


# speedup vs baseline: 5.1569x; 1.1910x over previous
import jax
import jax.numpy as jnp
from jax import lax
from jax.experimental import pallas as pl
from jax.experimental.pallas import tpu as pltpu
from jax.experimental.pallas import tpu_sc as plsc

NC = 2
NS = 16
L = 16

E = 320000
N = 10000
D = 128
G = 1024

HG = G // NC
EPT = E // NS
SCH = 2000
RCH = 64
PAD = 2 * RCH

_BCAST_DNUMS = lax.GatherDimensionNumbers(
    offset_dims=(), collapsed_slice_dims=(0,), start_index_map=(0,))


def _bcast_lane(vec, j):
    idx = jnp.full((L, 1), j, jnp.int32)
    return lax.gather(vec, idx, _BCAST_DNUMS, slice_sizes=(1,),
                      mode=lax.GatherScatterMode.PROMISE_IN_BOUNDS)


def _sc_body(edges_hbm, src_hbm, n2g_hbm, part_hbm,
             idx_c, ids_c, acc, ksmem, sem0, sem1):
    c = lax.axis_index("c")
    s = lax.axis_index("s")
    iota = lax.iota(jnp.int32, L)
    col_idx = [iota + kk * L for kk in range(D // L)]

    neg_inf = jnp.full((L,), -jnp.inf, jnp.float32)

    def init_row(r, _):
        for kk in range(D // L):
            acc[r, pl.ds(kk * L, L)] = neg_inf
        return 0

    lax.fori_loop(0, HG, init_row, 0)

    ebase = s * EPT
    cgbase = c * HG

    def phase_a(node_v, sbuf):
        pltpu.sync_copy(n2g_hbm, node_v)

        def chunk_a(i, k):
            base = ebase + i * SCH
            pltpu.sync_copy(src_hbm.at[pl.ds(base, SCH)], sbuf)

            def group(g, k):
                src_vec = sbuf[pl.ds(g * L, L)]
                ids = plsc.load_gather(node_v, [src_vec])
                ids = jnp.minimum(ids, G - 1) - cgbase
                m = (ids >= 0) & (ids < HG)
                mi = m.astype(jnp.int32)
                evec = jnp.full((L,), base + g * L, jnp.int32) + iota
                pos = k - 1 + plsc.cumsum(mi)
                plsc.store_scatter(idx_c, [pos], evec, mask=m)
                plsc.store_scatter(ids_c, [pos], ids, mask=m)
                return k + jnp.sum(mi)

            return lax.fori_loop(0, SCH // L, group, k)

        k = lax.fori_loop(0, EPT // SCH, chunk_a, jnp.int32(0))

        pad_ids = jnp.full((L,), HG, jnp.int32)
        zeros = jnp.zeros((L,), jnp.int32)
        for t in range(PAD // L):
            pos = k + t * L + iota
            plsc.store_scatter(idx_c, [pos], zeros)
            plsc.store_scatter(ids_c, [pos], pad_ids)
        ksmem[0] = jnp.bitwise_and(k + (PAD - 1), -PAD)

    pl.run_scoped(phase_a,
                  pltpu.VMEM((N,), jnp.int32),
                  pltpu.VMEM((SCH,), jnp.int32))

    kpad = ksmem[0]
    nchunks = jnp.right_shift(kpad, 6)

    def phase_b(rows0, rows1):
        rows = (rows0, rows1)
        sems = (sem0, sem1)

        def start(ch, b):
            pltpu.async_copy(edges_hbm.at[idx_c.at[pl.ds(ch * RCH, RCH)]],
                             rows[b], sems[b])

        def wait(ch, b):
            pltpu.make_async_copy(
                edges_hbm.at[idx_c.at[pl.ds(ch * RCH, RCH)]],
                rows[b], sems[b]).wait()

        start(jnp.int32(0), 0)

        def pair(p, _):
            for b in (0, 1):
                ch = 2 * p + b
                wait(ch, b)

                @pl.when(ch + 1 < nchunks)
                def _():
                    start(ch + 1, 1 - b)

                rv = rows[b]
                cb = ch * RCH

                def group(mq, _):
                    idvec = ids_c[pl.ds(cb + mq * L, L)]
                    for j in range(L):
                        gj = _bcast_lane(idvec, j)
                        row = mq * L + j
                        for kk in range(D // L):
                            a = plsc.load_gather(acc, [gj, col_idx[kk]])
                            e = rv[row, pl.ds(kk * L, L)]
                            plsc.store_scatter(acc, [gj, col_idx[kk]],
                                               jnp.maximum(a, e))
                    return 0

                lax.fori_loop(0, RCH // L, group, 0)
            return 0

        lax.fori_loop(0, jnp.right_shift(nchunks, 1), pair, 0)

    pl.run_scoped(phase_b,
                  pltpu.VMEM((RCH, D), jnp.float32),
                  pltpu.VMEM((RCH, D), jnp.float32))

    pltpu.sync_copy(acc.at[pl.ds(0, HG), :],
                    part_hbm.at[s, pl.ds(cgbase, HG), :])


def _segment_max_partials(edges, src, n2g):
    mesh = plsc.VectorSubcoreMesh(core_axis_name="c", subcore_axis_name="s")
    fn = pl.kernel(
        _sc_body,
        out_type=jax.ShapeDtypeStruct((NS, G, D), jnp.float32),
        mesh=mesh,
        scratch_types=[
            pltpu.VMEM((EPT + 2 * PAD,), jnp.int32),
            pltpu.VMEM((EPT + 2 * PAD,), jnp.int32),
            pltpu.VMEM((HG + 8, D), jnp.float32),
            pltpu.SMEM((1,), jnp.int32),
            pltpu.SemaphoreType.DMA,
            pltpu.SemaphoreType.DMA,
        ],
        compiler_params=pltpu.CompilerParams(needs_layout_passes=False),
    )
    return fn(edges, src, n2g)


def _mlp_body(p_ref, w0_ref, b0_ref, w1_ref, b1_ref, w2_ref, b2_ref, out_ref):
    x = jnp.max(p_ref[...], axis=0)
    x = jnp.where(x == -jnp.inf, 0.0, x)
    h = jnp.dot(x, w0_ref[...], preferred_element_type=jnp.float32,
                precision="highest") + b0_ref[...]
    h = jnp.maximum(h, 0.0)
    h = jnp.dot(h, w1_ref[...], preferred_element_type=jnp.float32,
                precision="highest") + b1_ref[...]
    h = jnp.maximum(h, 0.0)
    out_ref[...] = jnp.dot(h, w2_ref[...], preferred_element_type=jnp.float32,
                           precision="highest") + b2_ref[...]


def _mlp(partials, W0, b0, W1, b1, W2, b2):
    return pl.pallas_call(
        _mlp_body,
        out_shape=jax.ShapeDtypeStruct((G, W2.shape[1]), jnp.float32),
    )(partials, W0, b0.reshape(1, -1), W1, b1.reshape(1, -1),
      W2, b2.reshape(1, -1))


def kernel(edges, edge_indices, node_to_graph_idx, num_graphs,
           W0, b0, W1, b1, W2, b2):
    del num_graphs
    src = edge_indices[0]
    partials = _segment_max_partials(edges, src, node_to_graph_idx)
    return _mlp(partials, W0, b0, W1, b1, W2, b2)


# speedup vs baseline: 5.4053x; 1.0482x over previous
import jax
import jax.numpy as jnp
from jax import lax
from jax.experimental import pallas as pl
from jax.experimental.pallas import tpu as pltpu
from jax.experimental.pallas import tpu_sc as plsc

NC = 2
NS = 16
L = 16

E = 320000
N = 10000
D = 128
G = 1024

HG = G // NC
EPT = E // NS
NR = 2
EPR = EPT // NR
SCH = 2000
RCH = 64
NBUF = 4
PAD = NBUF * RCH

_BCAST_DNUMS = lax.GatherDimensionNumbers(
    offset_dims=(), collapsed_slice_dims=(0,), start_index_map=(0,))


def _bcast_lane(vec, j):
    idx = jnp.full((L, 1), j, jnp.int32)
    return lax.gather(vec, idx, _BCAST_DNUMS, slice_sizes=(1,),
                      mode=lax.GatherScatterMode.PROMISE_IN_BOUNDS)


def _sc_body(edges_hbm, src_hbm, n2g_hbm, part_hbm,
             idx_c, ids_c, acc,
             node_v, sbuf, rows0, rows1, rows2, rows3,
             ksmem, sem0, sem1, sem2, sem3):
    c = lax.axis_index("c")
    s = lax.axis_index("s")
    iota = lax.iota(jnp.int32, L)
    col_idx = [iota + kk * L for kk in range(D // L)]

    neg_inf = jnp.full((L,), -jnp.inf, jnp.float32)

    def init_row(r, _):
        for kk in range(D // L):
            acc[pl.ds(r * D + kk * L, L)] = neg_inf
        return 0

    with jax.named_scope("init"):
        lax.fori_loop(0, HG, init_row, 0)

    cgbase = c * HG
    pltpu.sync_copy(n2g_hbm, node_v)

    def phase_a(ebase):
        def chunk_a(i, k):
            base = ebase + i * SCH
            pltpu.sync_copy(src_hbm.at[pl.ds(base, SCH)], sbuf)

            def group(g, k):
                src_vec = sbuf[pl.ds(g * L, L)]
                word = plsc.load_gather(node_v, [jnp.right_shift(src_vec, 1)])
                amt = jnp.left_shift(jnp.bitwise_and(src_vec, 1), 4)
                ids = jnp.bitwise_and(
                    lax.shift_right_logical(word, amt), 0xFFFF)
                ids = jnp.minimum(ids, G - 1) - cgbase
                m = (ids >= 0) & (ids < HG)
                mi = m.astype(jnp.int32)
                evec = jnp.full((L,), base + g * L, jnp.int32) + iota
                pos = k - 1 + plsc.cumsum(mi)
                plsc.store_scatter(idx_c, [pos], evec, mask=m)
                plsc.store_scatter(ids_c, [pos], ids, mask=m)
                return k + jnp.sum(mi)

            return lax.fori_loop(0, SCH // L, group, k)

        k = lax.fori_loop(0, EPR // SCH, chunk_a, jnp.int32(0))

        last = jnp.full((L,), jnp.maximum(k - 1, 0), jnp.int32)
        pad_ids = plsc.load_gather(ids_c, [last])
        pad_idx = plsc.load_gather(idx_c, [last])
        for t in range(PAD // L):
            pos = k + t * L + iota
            plsc.store_scatter(idx_c, [pos], pad_idx)
            plsc.store_scatter(ids_c, [pos], pad_ids)
        return jnp.bitwise_and(k + (PAD - 1), -PAD)

    def phase_b(nchunks):
        rows = (rows0, rows1, rows2, rows3)
        sems = (sem0, sem1, sem2, sem3)

        def start(ch, b):
            pltpu.async_copy(edges_hbm.at[idx_c.at[pl.ds(ch * RCH, RCH)]],
                             rows[b], sems[b])

        def wait(ch, b):
            pltpu.make_async_copy(
                edges_hbm.at[idx_c.at[pl.ds(ch * RCH, RCH)]],
                rows[b], sems[b]).wait()

        for t in range(NBUF - 1):
            @pl.when(t < nchunks)
            def _():
                start(jnp.int32(t), t)

        def pair(p, _):
            for b in range(NBUF):
                ch = NBUF * p + b

                @pl.when(ch + NBUF - 1 < nchunks)
                def _():
                    start(ch + NBUF - 1, (b + NBUF - 1) % NBUF)

                wait(ch, b)
                rv = rows[b]
                cb = ch * RCH

                def group(mq, _):
                    idvec = ids_c[pl.ds(cb + mq * L, L)]
                    bases = jnp.left_shift(idvec, 7)
                    for j in range(L):
                        base = _bcast_lane(bases, j)
                        row = mq * L + j
                        for kk in range(D // L):
                            ix = base + col_idx[kk]
                            a = plsc.load_gather(acc, [ix])
                            e = rv[row, pl.ds(kk * L, L)]
                            plsc.store_scatter(acc, [ix],
                                               jnp.maximum(a, e))
                    return 0

                lax.fori_loop(0, RCH // L, group, 0)
            return 0

        lax.fori_loop(0, jnp.right_shift(nchunks, 2), pair, 0)

    for r in range(NR):
        with jax.named_scope("phaseA"):
            kpad = phase_a(s * EPT + r * EPR)
        with jax.named_scope("phaseB"):
            phase_b(jnp.right_shift(kpad, 6))

    pltpu.sync_copy(acc, part_hbm.at[s, pl.ds(cgbase * D, HG * D)])


def _segment_max_partials(edges, src, n2g):
    mesh = plsc.VectorSubcoreMesh(core_axis_name="c", subcore_axis_name="s")
    fn = pl.kernel(
        _sc_body,
        out_type=jax.ShapeDtypeStruct((NS, G * D), jnp.float32),
        mesh=mesh,
        scratch_types=[
            pltpu.VMEM((EPR + PAD,), jnp.int32),
            pltpu.VMEM((EPR + PAD,), jnp.int32),
            pltpu.VMEM((HG * D,), jnp.float32),
            pltpu.VMEM((N // 2,), jnp.int32),
            pltpu.VMEM((SCH,), jnp.int32),
            pltpu.VMEM((RCH, D), jnp.float32),
            pltpu.VMEM((RCH, D), jnp.float32),
            pltpu.VMEM((RCH, D), jnp.float32),
            pltpu.VMEM((RCH, D), jnp.float32),
            pltpu.SMEM((1,), jnp.int32),
            pltpu.SemaphoreType.DMA,
            pltpu.SemaphoreType.DMA,
            pltpu.SemaphoreType.DMA,
            pltpu.SemaphoreType.DMA,
        ],
        compiler_params=pltpu.CompilerParams(needs_layout_passes=False, disable_bounds_checks=True),
    )
    return fn(edges, src, n2g)


def _mlp_body(p_ref, w0_ref, b0_ref, w1_ref, b1_ref, w2_ref, b2_ref, out_ref):
    x = jnp.max(p_ref[...], axis=0)
    x = jnp.where(x == -jnp.inf, 0.0, x)
    h = jnp.dot(x, w0_ref[...], preferred_element_type=jnp.float32,
                precision="highest") + b0_ref[...]
    h = jnp.maximum(h, 0.0)
    h = jnp.dot(h, w1_ref[...], preferred_element_type=jnp.float32,
                precision="highest") + b1_ref[...]
    h = jnp.maximum(h, 0.0)
    out_ref[...] = jnp.dot(h, w2_ref[...], preferred_element_type=jnp.float32,
                           precision="highest") + b2_ref[...]


def _mlp(partials, W0, b0, W1, b1, W2, b2):
    return pl.pallas_call(
        _mlp_body,
        out_shape=jax.ShapeDtypeStruct((G, W2.shape[1]), jnp.float32),
    )(partials, W0, b0.reshape(1, -1), W1, b1.reshape(1, -1),
      W2, b2.reshape(1, -1))


def kernel(edges, edge_indices, node_to_graph_idx, num_graphs,
           W0, b0, W1, b1, W2, b2):
    del num_graphs
    src = edge_indices[0]
    pairs = node_to_graph_idx.reshape(N // 2, 2)
    n2g_packed = pairs[:, 0] | (pairs[:, 1] << 16)
    partials = _segment_max_partials(edges, src, n2g_packed)
    partials = partials.reshape(NS, G, D)
    return _mlp(partials, W0, b0, W1, b1, W2, b2)


# speedup vs baseline: 5.8247x; 1.0776x over previous
import jax
import jax.numpy as jnp
from jax import lax
from jax.experimental import pallas as pl
from jax.experimental.pallas import tpu as pltpu
from jax.experimental.pallas import tpu_sc as plsc

NC = 2
NS = 16
L = 16

E = 320000
N = 10000
D = 128
G = 1024

HG = G // NC
EPT = E // NS
NR = 2
EPR = EPT // NR
SCH = 2000
RCH = 64
PAD = 2 * RCH

_BCAST_DNUMS = lax.GatherDimensionNumbers(
    offset_dims=(), collapsed_slice_dims=(0,), start_index_map=(0,))


def _bcast_lane(vec, j):
    idx = jnp.full((L, 1), j, jnp.int32)
    return lax.gather(vec, idx, _BCAST_DNUMS, slice_sizes=(1,),
                      mode=lax.GatherScatterMode.PROMISE_IN_BOUNDS)


def _sc_body(edges_hbm, src_hbm, n2g_hbm, part_hbm,
             idx_c, ids_c, acc,
             node_v, sbuf, rows0, rows1, ksmem, sem0, sem1):
    c = lax.axis_index("c")
    s = lax.axis_index("s")
    iota = lax.iota(jnp.int32, L)
    col_idx = [iota + kk * L for kk in range(D // L)]

    neg_inf = jnp.full((L,), -jnp.inf, jnp.float32)

    def init_row(r, _):
        for kk in range(D // L):
            acc[r, pl.ds(kk * L, L)] = neg_inf
        return 0

    lax.fori_loop(0, HG, init_row, 0)

    cgbase = c * HG
    pltpu.sync_copy(n2g_hbm, node_v)

    def phase_a(ebase):
        def chunk_a(i, k):
            base = ebase + i * SCH
            pltpu.sync_copy(src_hbm.at[pl.ds(base, SCH)], sbuf)

            def group(g, k):
                src_vec = sbuf[pl.ds(g * L, L)]
                ids = plsc.load_gather(node_v, [src_vec])
                ids = jnp.minimum(ids, G - 1) - cgbase
                m = (ids >= 0) & (ids < HG)
                mi = m.astype(jnp.int32)
                evec = jnp.full((L,), base + g * L, jnp.int32) + iota
                pos = k - 1 + plsc.cumsum(mi)
                plsc.store_scatter(idx_c, [pos], evec, mask=m)
                plsc.store_scatter(ids_c, [pos], ids, mask=m)
                return k + jnp.sum(mi)

            return lax.fori_loop(0, SCH // L, group, k)

        k = lax.fori_loop(0, EPR // SCH, chunk_a, jnp.int32(0))

        last = jnp.full((L,), jnp.maximum(k - 1, 0), jnp.int32)
        pad_ids = plsc.load_gather(ids_c, [last])
        pad_idx = plsc.load_gather(idx_c, [last])
        for t in range(PAD // L):
            pos = k + t * L + iota
            plsc.store_scatter(idx_c, [pos], pad_idx)
            plsc.store_scatter(ids_c, [pos], pad_ids)
        return jnp.bitwise_and(k + (PAD - 1), -PAD)

    def phase_b(nchunks):
        rows = (rows0, rows1)
        sems = (sem0, sem1)

        def start(ch, b):
            pltpu.async_copy(edges_hbm.at[idx_c.at[pl.ds(ch * RCH, RCH)]],
                             rows[b], sems[b])

        def wait(ch, b):
            pltpu.make_async_copy(
                edges_hbm.at[idx_c.at[pl.ds(ch * RCH, RCH)]],
                rows[b], sems[b]).wait()

        start(jnp.int32(0), 0)

        def pair(p, _):
            for b in (0, 1):
                ch = 2 * p + b

                @pl.when(ch + 1 < nchunks)
                def _():
                    start(ch + 1, 1 - b)

                wait(ch, b)
                rv = rows[b]
                cb = ch * RCH

                def group(mq, _):
                    idvec = ids_c[pl.ds(cb + mq * L, L)]
                    for j in range(L):
                        gj = _bcast_lane(idvec, j)
                        row = mq * L + j
                        for kk in range(D // L):
                            a = plsc.load_gather(acc, [gj, col_idx[kk]])
                            e = rv[row, pl.ds(kk * L, L)]
                            plsc.store_scatter(acc, [gj, col_idx[kk]],
                                               jnp.maximum(a, e))
                    return 0

                lax.fori_loop(0, RCH // L, group, 0)
            return 0

        lax.fori_loop(0, jnp.right_shift(nchunks, 1), pair, 0)

    for r in range(NR):
        with jax.named_scope("phaseA"):
            kpad = phase_a(s * EPT + r * EPR)
        with jax.named_scope("phaseB"):
            phase_b(jnp.right_shift(kpad, 6))

    pltpu.sync_copy(acc, part_hbm.at[s, pl.ds(cgbase, HG), :])


def _segment_max_partials(edges, src, n2g):
    mesh = plsc.VectorSubcoreMesh(core_axis_name="c", subcore_axis_name="s")
    fn = pl.kernel(
        _sc_body,
        out_type=jax.ShapeDtypeStruct((NS, G, D), jnp.float32),
        mesh=mesh,
        scratch_types=[
            pltpu.VMEM((EPR + 2 * PAD,), jnp.int32),
            pltpu.VMEM((EPR + 2 * PAD,), jnp.int32),
            pltpu.VMEM((HG, D), jnp.float32),
            pltpu.VMEM((N,), jnp.int32),
            pltpu.VMEM((SCH,), jnp.int32),
            pltpu.VMEM((RCH, D), jnp.float32),
            pltpu.VMEM((RCH, D), jnp.float32),
            pltpu.SMEM((1,), jnp.int32),
            pltpu.SemaphoreType.DMA,
            pltpu.SemaphoreType.DMA,
        ],
        compiler_params=pltpu.CompilerParams(needs_layout_passes=False),
    )
    return fn(edges, src, n2g)


def _mlp_body(p_ref, w0_ref, b0_ref, w1_ref, b1_ref, w2_ref, b2_ref, out_ref):
    x = jnp.max(p_ref[...], axis=0)
    x = jnp.where(x == -jnp.inf, 0.0, x)
    h = jnp.dot(x, w0_ref[...], preferred_element_type=jnp.float32,
                precision="highest") + b0_ref[...]
    h = jnp.maximum(h, 0.0)
    h = jnp.dot(h, w1_ref[...], preferred_element_type=jnp.float32,
                precision="highest") + b1_ref[...]
    h = jnp.maximum(h, 0.0)
    out_ref[...] = jnp.dot(h, w2_ref[...], preferred_element_type=jnp.float32,
                           precision="highest") + b2_ref[...]


def _mlp(partials, W0, b0, W1, b1, W2, b2):
    return pl.pallas_call(
        _mlp_body,
        out_shape=jax.ShapeDtypeStruct((G, W2.shape[1]), jnp.float32),
    )(partials, W0, b0.reshape(1, -1), W1, b1.reshape(1, -1),
      W2, b2.reshape(1, -1))


def kernel(edges, edge_indices, node_to_graph_idx, num_graphs,
           W0, b0, W1, b1, W2, b2):
    del num_graphs
    src = edge_indices[0]
    partials = _segment_max_partials(edges, src, node_to_graph_idx)
    return _mlp(partials, W0, b0, W1, b1, W2, b2)
